# baseline (speedup 1.0000x reference)
import functools

import jax
import jax.numpy as jnp
from jax import lax
from jax.experimental import pallas as pl
from jax.experimental.pallas import tpu as pltpu
from jax.experimental.pallas import tpu_sc as plsc

D_MODEL = 64
SCALE = 8.0
D_PAD = 128

NC = 2
NS = 16
NW = NC * NS

G = 2
GW = 4
LANES = 16

V_BLK = 8192


def _transpose_kernel(tab_ref, out_ref):
    out_ref[:, :D_MODEL] = jnp.swapaxes(tab_ref[...], 0, 1) * SCALE


NBUF = 2


def _gather_kernel(n_grp, seq, idx_hbm, table_hbm, out_hbm, idx_v,
                   rows0, rows1, out0, out1, gsem0, gsem1, wsem0, wsem1):
    rows = (rows0, rows1)
    outs = (out0, out1)
    gsems = (gsem0, gsem1)
    wsems = (wsem0, wsem1)
    nsub = GW // G
    n_sup = n_grp // nsub
    wid = lax.axis_index("s") * NC + lax.axis_index("c")
    t_base = wid * n_grp * G
    pltpu.sync_copy(idx_hbm.at[wid], idx_v)

    for b in range(NBUF):
        pltpu.async_copy(table_hbm.at[idx_v.at[b]], rows[b], gsems[b])

    @pl.loop(0, n_sup, step=2)
    def _(sup0):
        for o in range(2):
            sup = sup0 + o
            @pl.when(sup >= 2)
            def _():
                pltpu.make_async_copy(
                    outs[o],
                    out_hbm.at[pl.ds((t_base + (sup - 2) * GW) * seq,
                                     GW * seq)],
                    wsems[o]).wait()

            for q in range(nsub):
                b = (o * nsub + q) % NBUF
                jj = sup * nsub + q
                pltpu.make_async_copy(
                    table_hbm.at[idx_v.at[jj]], rows[b], gsems[b]).wait()

                @pl.loop(0, G)
                def _(g):
                    @pl.loop(0, seq)
                    def _(s):
                        for c in range(D_MODEL // LANES):
                            cs = pl.ds(c * LANES, LANES)
                            outs[o].at[(q * G + g) * seq + s, cs][...] = (
                                rows[b].at[g * seq + s, cs][...])

                @pl.when(jj + NBUF < n_grp)
                def _():
                    pltpu.async_copy(
                        table_hbm.at[idx_v.at[jj + NBUF]], rows[b], gsems[b])

            pltpu.async_copy(
                outs[o],
                out_hbm.at[pl.ds((t_base + sup * GW) * seq, GW * seq)],
                wsems[o])

    for o in range(2):
        pltpu.make_async_copy(
            outs[o],
            out_hbm.at[pl.ds((t_base + (n_sup - 2 + o) * GW) * seq,
                             GW * seq)],
            wsems[o]).wait()


@jax.jit
def kernel(x, table):
    n_tokens, seq = x.shape
    assert n_tokens % (NW * G) == 0
    n_grp = n_tokens // (NW * G)
    vocab = table.shape[0]
    n_blk = -(-vocab // V_BLK)

    idx = x.reshape(NW, n_grp, G * seq).astype(jnp.int32)
    tab_t = jnp.swapaxes(table, 0, 1)

    tabp = pl.pallas_call(
        _transpose_kernel,
        grid=(n_blk,),
        in_specs=[pl.BlockSpec((D_MODEL, V_BLK), lambda i: (0, i))],
        out_specs=pl.BlockSpec((V_BLK, D_PAD), lambda i: (i, 0)),
        out_shape=jax.ShapeDtypeStruct((vocab, D_PAD), jnp.float32),
    )(tab_t)

    mesh = plsc.VectorSubcoreMesh(core_axis_name="c", subcore_axis_name="s")
    run = pl.kernel(
        functools.partial(_gather_kernel, n_grp, seq),
        out_type=jax.ShapeDtypeStruct((n_tokens * seq, D_MODEL), jnp.float32),
        mesh=mesh,
        compiler_params=pltpu.CompilerParams(use_tc_tiling_on_sc=True),
        scratch_types=[
            pltpu.VMEM((n_grp, G * seq), jnp.int32),
            pltpu.VMEM((G * seq, D_PAD), jnp.float32),
            pltpu.VMEM((G * seq, D_PAD), jnp.float32),
            pltpu.VMEM((GW * seq, D_MODEL), jnp.float32),
            pltpu.VMEM((GW * seq, D_MODEL), jnp.float32),
            pltpu.SemaphoreType.DMA,
            pltpu.SemaphoreType.DMA,
            pltpu.SemaphoreType.DMA,
            pltpu.SemaphoreType.DMA,
        ],
    )
    return run(idx, tabp).reshape(n_tokens, seq, D_MODEL)


# speedup vs baseline: 1.4025x; 1.4025x over previous
import functools

import jax
import jax.numpy as jnp
from jax import lax
from jax.experimental import pallas as pl
from jax.experimental.pallas import tpu as pltpu
from jax.experimental.pallas import tpu_sc as plsc

D_MODEL = 64
SCALE = 8.0
D_PAD = 128

NC = 2
NS = 16
NW = NC * NS

G = 2
LANES = 16

V_BLK = 8192


def _transpose_kernel(tab_ref, out_ref):
    out_ref[:, :D_MODEL] = jnp.swapaxes(tab_ref[...], 0, 1) * SCALE


NBUF = 2


def _gather_kernel(n_grp, seq, idx_hbm, table_hbm, out_hbm, idx_v,
                   rows0, rows1, out0, out1, gsem0, gsem1, wsem0, wsem1):
    rows = (rows0, rows1)
    outs = (out0, out1)
    gsems = (gsem0, gsem1)
    wsems = (wsem0, wsem1)
    wid = lax.axis_index("s") * NC + lax.axis_index("c")
    t_base = wid * n_grp * G
    pltpu.sync_copy(idx_hbm.at[wid], idx_v)

    def slab(j):
        return out_hbm.at[pl.ds(t_base + j * G, G)]

    for b in range(NBUF):
        pltpu.async_copy(table_hbm.at[idx_v.at[b]], rows[b], gsems[b])

    @pl.loop(0, n_grp, step=NBUF)
    def _(j):
        for b in range(NBUF):
            jj = j + b
            pltpu.make_async_copy(
                table_hbm.at[idx_v.at[jj]], rows[b], gsems[b]).wait()
            @pl.when(jj >= NBUF)
            def _():
                pltpu.make_async_copy(outs[b], slab(jj), wsems[b]).wait()

            @pl.loop(0, G)
            def _(g):
                for s in range(seq):
                    for c in range(D_MODEL // LANES):
                        outs[b].at[g, pl.ds(s * D_MODEL + c * LANES,
                                            LANES)][...] = (
                            rows[b].at[g * seq + s,
                                       pl.ds(c * LANES, LANES)][...])

            @pl.when(jj + NBUF < n_grp)
            def _():
                pltpu.async_copy(
                    table_hbm.at[idx_v.at[jj + NBUF]], rows[b], gsems[b])
            pltpu.async_copy(outs[b], slab(jj), wsems[b])

    for b in range(NBUF):
        pltpu.make_async_copy(
            outs[b], slab(n_grp - NBUF + b), wsems[b]).wait()


@jax.jit
def kernel(x, table):
    n_tokens, seq = x.shape
    assert n_tokens % (NW * G) == 0
    n_grp = n_tokens // (NW * G)
    vocab = table.shape[0]
    n_blk = -(-vocab // V_BLK)

    idx = x.reshape(NW, n_grp, G * seq).astype(jnp.int32)
    tab_t = jnp.swapaxes(table, 0, 1)

    tabp = pl.pallas_call(
        _transpose_kernel,
        grid=(n_blk,),
        in_specs=[pl.BlockSpec((D_MODEL, V_BLK), lambda i: (0, i))],
        out_specs=pl.BlockSpec((V_BLK, D_PAD), lambda i: (i, 0)),
        out_shape=jax.ShapeDtypeStruct((vocab, D_PAD), jnp.float32),
    )(tab_t)

    mesh = plsc.VectorSubcoreMesh(core_axis_name="c", subcore_axis_name="s")
    run = pl.kernel(
        functools.partial(_gather_kernel, n_grp, seq),
        out_type=jax.ShapeDtypeStruct((n_tokens, seq * D_MODEL), jnp.float32),
        mesh=mesh,
        compiler_params=pltpu.CompilerParams(use_tc_tiling_on_sc=True),
        scratch_types=[
            pltpu.VMEM((n_grp, G * seq), jnp.int32),
            pltpu.VMEM((G * seq, D_PAD), jnp.float32),
            pltpu.VMEM((G * seq, D_PAD), jnp.float32),
            pltpu.VMEM((G, seq * D_MODEL), jnp.float32),
            pltpu.VMEM((G, seq * D_MODEL), jnp.float32),
            pltpu.SemaphoreType.DMA,
            pltpu.SemaphoreType.DMA,
            pltpu.SemaphoreType.DMA,
            pltpu.SemaphoreType.DMA,
        ],
    )
    return run(idx, tabp).reshape(n_tokens, seq, D_MODEL)
